# baseline (speedup 1.0000x reference)
import functools

import jax
import jax.numpy as jnp
from jax import lax
from jax.experimental import pallas as pl
from jax.experimental.pallas import tpu as pltpu
from jax.experimental.pallas import tpu_sc as plsc

_N = 10000
_D = 128
_E = 320000
_G = 64

_NC = 2
_NS = 16
_NW = _NC * _NS
_EPW = _E // _NW
_CHUNK = 40
_GRP = 2000
_NGRP = _EPW // _GRP
_GCH = _GRP // _CHUNK
_GPAIR = _GCH // 2
_RPS = 624
_TAIL = _N - _NS * _RPS
_ZR = 16
_BM = 1000


def _msg_body(x_hbm, src_hbm, dst_hbm, ea_hbm, w_hbm, agg_hbm,
              src_v, dst_v, attr_v, w_v, rows_a, rows_b, zero_v, agg_sh,
              g_a, g_b, s_a, s_b):
    c = lax.axis_index("c")
    s = lax.axis_index("s")
    wid = c * _NS + s

    pltpu.sync_copy(w_hbm, w_v)

    zvec = jnp.zeros((16,), jnp.float32)

    def _zrow(r, carry):
        for k in range(8):
            zero_v[r, pl.ds(k * 16, 16)] = zvec
        return carry

    lax.fori_loop(0, _ZR, _zrow, 0)

    def _zcopy(t, carry):
        pltpu.sync_copy(zero_v, agg_sh.at[pl.ds(s * _RPS + t * _ZR, _ZR)])
        return carry

    lax.fori_loop(0, _RPS // _ZR, _zcopy, 0)

    @pl.when(s == _NS - 1)
    def _():
        for t in range(_TAIL // _ZR):
            pltpu.sync_copy(zero_v,
                            agg_sh.at[pl.ds(_NS * _RPS + t * _ZR, _ZR)])

    plsc.subcore_barrier()

    wvec = [[w_v[r, pl.ds(k * 16, 16)] for k in range(8)] for r in range(4)]

    def _gather_start(q, rows, sem):
        pltpu.async_copy(x_hbm.at[src_v.at[pl.ds(q * _CHUNK, _CHUNK)]],
                         rows, sem)

    def _gather_wait(q, rows, sem):
        pltpu.make_async_copy(x_hbm.at[src_v.at[pl.ds(q * _CHUNK, _CHUNK)]],
                              rows, sem).wait()

    def _scat_start(q, rows, sem):
        pltpu.async_copy(rows, agg_sh.at[dst_v.at[q]], sem, add=True)

    def _scat_wait(q, rows, sem):
        pltpu.make_async_copy(rows, agg_sh.at[dst_v.at[q]], sem).wait()

    def _compute(q, rows):
        base3 = q * (_CHUNK * 3)

        @plsc.parallel_loop(0, _CHUNK, 1, unroll=3)
        def _edge(e):
            av = attr_v[pl.ds(base3 + e * 3, 16)]
            a0 = av[0]
            a1 = av[1]
            a2 = av[2]
            for k in range(8):
                row = rows[e, pl.ds(k * 16, 16)]
                ea = wvec[3][k] + a0 * wvec[0][k]
                ea = ea + a1 * wvec[1][k]
                ea = ea + a2 * wvec[2][k]
                rows[e, pl.ds(k * 16, 16)] = jnp.maximum(row + ea, 0.0)

    def _group(g, carry):
        gbase = wid * _EPW + g * _GRP
        pltpu.sync_copy(src_hbm.at[pl.ds(gbase, _GRP)], src_v)
        pltpu.sync_copy(dst_hbm.at[wid, g], dst_v)
        pltpu.sync_copy(ea_hbm.at[pl.ds(gbase * 3, _GRP * 3)],
                        attr_v.at[pl.ds(0, _GRP * 3)])
        _gather_start(0, rows_a, g_a)

        def _pair(i, icarry):
            q0 = 2 * i
            q1 = q0 + 1

            @pl.when(i > 0)
            def _():
                _scat_wait(q0 - 1, rows_b, s_b)

            _gather_start(q1, rows_b, g_b)
            _gather_wait(q0, rows_a, g_a)
            _compute(q0, rows_a)
            _scat_start(q0, rows_a, s_a)
            _scat_wait(q0, rows_a, s_a)

            @pl.when(i < _GPAIR - 1)
            def _():
                _gather_start(q0 + 2, rows_a, g_a)

            _gather_wait(q1, rows_b, g_b)
            _compute(q1, rows_b)
            _scat_start(q1, rows_b, s_b)
            return icarry

        lax.fori_loop(0, _GPAIR, _pair, 0)
        _scat_wait(_GCH - 1, rows_b, s_b)
        return carry

    lax.fori_loop(0, _NGRP, _group, 0)
    plsc.subcore_barrier()
    pltpu.sync_copy(agg_sh.at[pl.ds(s * _RPS, _RPS)],
                    agg_hbm.at[c, pl.ds(s * _RPS, _RPS)])

    @pl.when(s == _NS - 1)
    def _():
        pltpu.sync_copy(agg_sh.at[pl.ds(_NS * _RPS, _TAIL)],
                        agg_hbm.at[c, pl.ds(_NS * _RPS, _TAIL)])


_msg = pl.kernel(
    _msg_body,
    out_type=jax.ShapeDtypeStruct((_NC, _N, _D), jnp.float32),
    mesh=plsc.VectorSubcoreMesh(core_axis_name="c", subcore_axis_name="s"),
    scratch_types=[
        pltpu.VMEM((_GRP,), jnp.int32),
        pltpu.VMEM((_GCH, _CHUNK), jnp.int32),
        pltpu.VMEM((_GRP * 3 + 16,), jnp.float32),
        pltpu.VMEM((4, _D), jnp.float32),
        pltpu.VMEM((_CHUNK, _D), jnp.float32),
        pltpu.VMEM((_CHUNK, _D), jnp.float32),
        pltpu.VMEM((_ZR, _D), jnp.float32),
        pltpu.VMEM_SHARED((_N, _D), jnp.float32),
        pltpu.SemaphoreType.DMA,
        pltpu.SemaphoreType.DMA,
        pltpu.SemaphoreType.DMA,
        pltpu.SemaphoreType.DMA,
    ],
)


def _mlp_body(x_ref, a0_ref, a1_ref, wa_ref, ba_ref, wb_ref, bb_ref, o_ref):
    t = x_ref[...] + a0_ref[...] + a1_ref[...]
    u = jnp.dot(t, wa_ref[...], preferred_element_type=jnp.float32)
    u = jnp.maximum(u + ba_ref[...], 0.0)
    h = jnp.dot(u, wb_ref[...], preferred_element_type=jnp.float32)
    o_ref[...] = jnp.maximum(h + bb_ref[...], 0.0)


def _mlp(x, a0, a1, wa, ba, wb, bb):
    return pl.pallas_call(
        _mlp_body,
        grid=(_N // _BM,),
        in_specs=[
            pl.BlockSpec((_BM, _D), lambda i: (i, 0)),
            pl.BlockSpec((_BM, _D), lambda i: (i, 0)),
            pl.BlockSpec((_BM, _D), lambda i: (i, 0)),
            pl.BlockSpec((_D, _D), lambda i: (0, 0)),
            pl.BlockSpec((1, _D), lambda i: (0, 0)),
            pl.BlockSpec((_D, _D), lambda i: (0, 0)),
            pl.BlockSpec((1, _D), lambda i: (0, 0)),
        ],
        out_specs=pl.BlockSpec((_BM, _D), lambda i: (i, 0)),
        out_shape=jax.ShapeDtypeStruct((_N, _D), jnp.float32),
    )(x, a0, a1, wa, ba, wb, bb)


def _mlp_pool_body(h_ref, a0_ref, a1_ref, wa_ref, ba_ref, wb_ref, bb_ref,
                   bat_ref, o_ref, s_acc, c_acc):
    i = pl.program_id(0)
    t = h_ref[...] + a0_ref[...] + a1_ref[...]
    u = jnp.dot(t, wa_ref[...], preferred_element_type=jnp.float32)
    u = jnp.maximum(u + ba_ref[...], 0.0)
    h2 = jnp.dot(u, wb_ref[...], preferred_element_type=jnp.float32)
    h2 = jnp.maximum(h2 + bb_ref[...], 0.0)
    onehot = (bat_ref[...] == lax.broadcasted_iota(jnp.int32, (1, _G), 1))
    onehot = onehot.astype(jnp.float32)
    s_part = lax.dot_general(onehot, h2, (((0,), (0,)), ((), ())),
                             preferred_element_type=jnp.float32)
    c_part = lax.dot_general(onehot, jnp.ones((_BM, _D), jnp.float32),
                             (((0,), (0,)), ((), ())),
                             preferred_element_type=jnp.float32)

    @pl.when(i == 0)
    def _():
        s_acc[...] = jnp.zeros_like(s_acc)
        c_acc[...] = jnp.zeros_like(c_acc)

    s_acc[...] += s_part
    c_acc[...] += c_part

    @pl.when(i == pl.num_programs(0) - 1)
    def _():
        o_ref[...] = s_acc[...] / jnp.maximum(c_acc[...], 1.0)


def _mlp_pool(h, a0, a1, wa, ba, wb, bb, batch2):
    return pl.pallas_call(
        _mlp_pool_body,
        grid=(_N // _BM,),
        in_specs=[
            pl.BlockSpec((_BM, _D), lambda i: (i, 0)),
            pl.BlockSpec((_BM, _D), lambda i: (i, 0)),
            pl.BlockSpec((_BM, _D), lambda i: (i, 0)),
            pl.BlockSpec((_D, _D), lambda i: (0, 0)),
            pl.BlockSpec((1, _D), lambda i: (0, 0)),
            pl.BlockSpec((_D, _D), lambda i: (0, 0)),
            pl.BlockSpec((1, _D), lambda i: (0, 0)),
            pl.BlockSpec((_BM, 1), lambda i: (i, 0)),
        ],
        out_specs=pl.BlockSpec((_G, _D), lambda i: (0, 0)),
        out_shape=jax.ShapeDtypeStruct((_G, _D), jnp.float32),
        scratch_shapes=[
            pltpu.VMEM((_G, _D), jnp.float32),
            pltpu.VMEM((_G, _D), jnp.float32),
        ],
    )(h, a0, a1, wa, ba, wb, bb, batch2)


def kernel(x, edge_index, edge_attr, batch,
           W1e, b1e, W1a, b1a, W1b, b1b,
           W2e, b2e, W2a, b2a, W2b, b2b):
    src = edge_index[0].astype(jnp.int32)
    dst = edge_index[1].astype(jnp.int32).reshape(_NW, _NGRP, _GCH, _CHUNK)
    w1 = jnp.concatenate([W1e, b1e[None, :]], axis=0)
    w2 = jnp.concatenate([W2e, b2e[None, :]], axis=0)
    batch2 = batch.reshape(_N, 1).astype(jnp.int32)

    ea_flat = edge_attr.reshape(-1)

    agg1 = _msg(x, src, dst, ea_flat, w1)
    h1 = _mlp(x, agg1[0], agg1[1], W1a, b1a.reshape(1, _D),
              W1b, b1b.reshape(1, _D))
    agg2 = _msg(h1, src, dst, ea_flat, w2)
    return _mlp_pool(h1, agg2[0], agg2[1], W2a, b2a.reshape(1, _D),
                     W2b, b2b.reshape(1, _D), batch2)


# speedup vs baseline: 1.2549x; 1.2549x over previous
import functools

import jax
import jax.numpy as jnp
from jax import lax
from jax.experimental import pallas as pl
from jax.experimental.pallas import tpu as pltpu
from jax.experimental.pallas import tpu_sc as plsc

_N = 10000
_D = 128
_E = 320000
_G = 64

_NC = 2
_NS = 16
_NW = _NC * _NS
_EPW = _E // _NW
_CHUNK = 40
_GRP = 2000
_NGRP = _EPW // _GRP
_GCH = _GRP // _CHUNK
_GPAIR = _GCH // 2
_RPS = 624
_TAIL = _N - _NS * _RPS
_ZR = 16
_BM = 1000


def _msg_body(x_hbm, src_hbm, dst_hbm, ea_hbm, w_hbm, agg_hbm,
              src_v, dst_v, attr_v, w_v, rows_a, rows_b, dch_a, dch_b,
              zero_v, agg_sh, g_a, g_b, s_a, s_b):
    c = lax.axis_index("c")
    s = lax.axis_index("s")
    wid = c * _NS + s

    pltpu.sync_copy(w_hbm, w_v)

    zvec = jnp.zeros((16,), jnp.float32)

    def _zrow(r, carry):
        for k in range(8):
            zero_v[r, pl.ds(k * 16, 16)] = zvec
        return carry

    lax.fori_loop(0, _ZR, _zrow, 0)

    def _zcopy(t, carry):
        pltpu.sync_copy(zero_v, agg_sh.at[pl.ds(s * _RPS + t * _ZR, _ZR)])
        return carry

    lax.fori_loop(0, _RPS // _ZR, _zcopy, 0)

    @pl.when(s == _NS - 1)
    def _():
        for t in range(_TAIL // _ZR):
            pltpu.sync_copy(zero_v,
                            agg_sh.at[pl.ds(_NS * _RPS + t * _ZR, _ZR)])

    plsc.subcore_barrier()

    wvec = [[w_v[r, pl.ds(k * 16, 16)] for k in range(8)] for r in range(4)]

    def _gather_start(q, rows, sem):
        pltpu.async_copy(x_hbm.at[src_v.at[pl.ds(q * _CHUNK, _CHUNK)]],
                         rows, sem)

    def _gather_wait(q, rows, sem):
        pltpu.make_async_copy(x_hbm.at[src_v.at[pl.ds(q * _CHUNK, _CHUNK)]],
                              rows, sem).wait()

    def _scat_start(q, rows, sem, dch):
        dch[pl.ds(0, 16)] = dst_v[pl.ds(q * _CHUNK, 16)]
        dch[pl.ds(16, 16)] = dst_v[pl.ds(q * _CHUNK + 16, 16)]
        dch[pl.ds(24, 16)] = dst_v[pl.ds(q * _CHUNK + 24, 16)]
        pltpu.async_copy(rows, agg_sh.at[dch], sem, add=True)

    def _scat_wait(rows, sem, dch):
        pltpu.make_async_copy(rows, agg_sh.at[dch], sem).wait()

    def _compute(q, rows):
        base3 = q * (_CHUNK * 3)

        @plsc.parallel_loop(0, _CHUNK, 1, unroll=2)
        def _edge(e):
            av = attr_v[pl.ds(base3 + e * 3, 16)]
            a0 = av[0]
            a1 = av[1]
            a2 = av[2]
            for k in range(8):
                row = rows[e, pl.ds(k * 16, 16)]
                ea = wvec[3][k] + a0 * wvec[0][k]
                ea = ea + a1 * wvec[1][k]
                ea = ea + a2 * wvec[2][k]
                rows[e, pl.ds(k * 16, 16)] = jnp.maximum(row + ea, 0.0)

    def _group(g, carry):
        gbase = wid * _EPW + g * _GRP
        pltpu.sync_copy(src_hbm.at[pl.ds(gbase, _GRP)], src_v)
        pltpu.sync_copy(dst_hbm.at[pl.ds(gbase, _GRP)], dst_v)
        pltpu.sync_copy(ea_hbm.at[pl.ds(gbase * 3, _GRP * 3)],
                        attr_v.at[pl.ds(0, _GRP * 3)])
        _gather_start(0, rows_a, g_a)

        def _pair(i, icarry):
            q0 = 2 * i
            q1 = q0 + 1

            @pl.when(i > 0)
            def _():
                _scat_wait(rows_b, s_b, dch_b)

            _gather_start(q1, rows_b, g_b)
            _gather_wait(q0, rows_a, g_a)
            _compute(q0, rows_a)
            _scat_start(q0, rows_a, s_a, dch_a)
            _scat_wait(rows_a, s_a, dch_a)

            @pl.when(i < _GPAIR - 1)
            def _():
                _gather_start(q0 + 2, rows_a, g_a)

            _gather_wait(q1, rows_b, g_b)
            _compute(q1, rows_b)
            _scat_start(q1, rows_b, s_b, dch_b)
            return icarry

        lax.fori_loop(0, _GPAIR, _pair, 0)
        _scat_wait(rows_b, s_b, dch_b)
        return carry

    lax.fori_loop(0, _NGRP, _group, 0)
    plsc.subcore_barrier()
    pltpu.sync_copy(agg_sh.at[pl.ds(s * _RPS, _RPS)],
                    agg_hbm.at[c, pl.ds(s * _RPS, _RPS)])

    @pl.when(s == _NS - 1)
    def _():
        pltpu.sync_copy(agg_sh.at[pl.ds(_NS * _RPS, _TAIL)],
                        agg_hbm.at[c, pl.ds(_NS * _RPS, _TAIL)])


_msg = pl.kernel(
    _msg_body,
    out_type=jax.ShapeDtypeStruct((_NC, _N, _D), jnp.float32),
    mesh=plsc.VectorSubcoreMesh(core_axis_name="c", subcore_axis_name="s"),
    scratch_types=[
        pltpu.VMEM((_GRP,), jnp.int32),
        pltpu.VMEM((_GRP,), jnp.int32),
        pltpu.VMEM((_GRP * 3 + 16,), jnp.float32),
        pltpu.VMEM((4, _D), jnp.float32),
        pltpu.VMEM((_CHUNK, _D), jnp.float32),
        pltpu.VMEM((_CHUNK, _D), jnp.float32),
        pltpu.VMEM((_CHUNK,), jnp.int32),
        pltpu.VMEM((_CHUNK,), jnp.int32),
        pltpu.VMEM((_ZR, _D), jnp.float32),
        pltpu.VMEM_SHARED((_N, _D), jnp.float32),
        pltpu.SemaphoreType.DMA,
        pltpu.SemaphoreType.DMA,
        pltpu.SemaphoreType.DMA,
        pltpu.SemaphoreType.DMA,
    ],
)


def _mlp_body(x_ref, a0_ref, a1_ref, wa_ref, ba_ref, wb_ref, bb_ref, o_ref):
    t = x_ref[...] + a0_ref[0] + a1_ref[0]
    u = jnp.dot(t, wa_ref[...], preferred_element_type=jnp.float32)
    u = jnp.maximum(u + ba_ref[...], 0.0)
    h = jnp.dot(u, wb_ref[...], preferred_element_type=jnp.float32)
    o_ref[...] = jnp.maximum(h + bb_ref[...], 0.0)


def _mlp(x, agg, wa, ba, wb, bb):
    return pl.pallas_call(
        _mlp_body,
        grid=(_N // _BM,),
        in_specs=[
            pl.BlockSpec((_BM, _D), lambda i: (i, 0)),
            pl.BlockSpec((1, _BM, _D), lambda i: (0, i, 0)),
            pl.BlockSpec((1, _BM, _D), lambda i: (1, i, 0)),
            pl.BlockSpec((_D, _D), lambda i: (0, 0)),
            pl.BlockSpec((1, _D), lambda i: (0, 0)),
            pl.BlockSpec((_D, _D), lambda i: (0, 0)),
            pl.BlockSpec((1, _D), lambda i: (0, 0)),
        ],
        out_specs=pl.BlockSpec((_BM, _D), lambda i: (i, 0)),
        out_shape=jax.ShapeDtypeStruct((_N, _D), jnp.float32),
    )(x, agg, agg, wa, ba, wb, bb)


def _mlp_pool_body(h_ref, a0_ref, a1_ref, wa_ref, ba_ref, wb_ref, bb_ref,
                   bat_ref, o_ref, s_acc, c_acc):
    i = pl.program_id(0)
    t = h_ref[...] + a0_ref[0] + a1_ref[0]
    u = jnp.dot(t, wa_ref[...], preferred_element_type=jnp.float32)
    u = jnp.maximum(u + ba_ref[...], 0.0)
    h2 = jnp.dot(u, wb_ref[...], preferred_element_type=jnp.float32)
    h2 = jnp.maximum(h2 + bb_ref[...], 0.0)
    onehot = (bat_ref[...] == lax.broadcasted_iota(jnp.int32, (1, _G), 1))
    onehot = onehot.astype(jnp.float32)
    s_part = lax.dot_general(onehot, h2, (((0,), (0,)), ((), ())),
                             preferred_element_type=jnp.float32)
    c_part = lax.dot_general(onehot, jnp.ones((_BM, _D), jnp.float32),
                             (((0,), (0,)), ((), ())),
                             preferred_element_type=jnp.float32)

    @pl.when(i == 0)
    def _():
        s_acc[...] = jnp.zeros_like(s_acc)
        c_acc[...] = jnp.zeros_like(c_acc)

    s_acc[...] += s_part
    c_acc[...] += c_part

    @pl.when(i == pl.num_programs(0) - 1)
    def _():
        o_ref[...] = s_acc[...] / jnp.maximum(c_acc[...], 1.0)


def _mlp_pool(h, agg, wa, ba, wb, bb, batch2):
    return pl.pallas_call(
        _mlp_pool_body,
        grid=(_N // _BM,),
        in_specs=[
            pl.BlockSpec((_BM, _D), lambda i: (i, 0)),
            pl.BlockSpec((1, _BM, _D), lambda i: (0, i, 0)),
            pl.BlockSpec((1, _BM, _D), lambda i: (1, i, 0)),
            pl.BlockSpec((_D, _D), lambda i: (0, 0)),
            pl.BlockSpec((1, _D), lambda i: (0, 0)),
            pl.BlockSpec((_D, _D), lambda i: (0, 0)),
            pl.BlockSpec((1, _D), lambda i: (0, 0)),
            pl.BlockSpec((_BM, 1), lambda i: (i, 0)),
        ],
        out_specs=pl.BlockSpec((_G, _D), lambda i: (0, 0)),
        out_shape=jax.ShapeDtypeStruct((_G, _D), jnp.float32),
        scratch_shapes=[
            pltpu.VMEM((_G, _D), jnp.float32),
            pltpu.VMEM((_G, _D), jnp.float32),
        ],
    )(h, agg, agg, wa, ba, wb, bb, batch2)


def kernel(x, edge_index, edge_attr, batch,
           W1e, b1e, W1a, b1a, W1b, b1b,
           W2e, b2e, W2a, b2a, W2b, b2b):
    src = edge_index[0].astype(jnp.int32)
    dst = edge_index[1].astype(jnp.int32)
    w1 = jnp.concatenate([W1e, b1e[None, :]], axis=0)
    w2 = jnp.concatenate([W2e, b2e[None, :]], axis=0)
    batch2 = batch.reshape(_N, 1).astype(jnp.int32)

    ea_flat = edge_attr.reshape(-1)

    agg1 = _msg(x, src, dst, ea_flat, w1)
    h1 = _mlp(x, agg1, W1a, b1a.reshape(1, _D), W1b, b1b.reshape(1, _D))
    agg2 = _msg(h1, src, dst, ea_flat, w2)
    return _mlp_pool(h1, agg2, W2a, b2a.reshape(1, _D),
                     W2b, b2b.reshape(1, _D), batch2)


# speedup vs baseline: 1.2554x; 1.0004x over previous
import functools

import jax
import jax.numpy as jnp
from jax import lax
from jax.experimental import pallas as pl
from jax.experimental.pallas import tpu as pltpu
from jax.experimental.pallas import tpu_sc as plsc

_N = 10000
_D = 128
_E = 320000
_G = 64

_NC = 2
_NS = 16
_NW = _NC * _NS
_EPW = _E // _NW
_CHUNK = 40
_GRP = 2000
_NGRP = _EPW // _GRP
_GCH = _GRP // _CHUNK
_GPAIR = _GCH // 2
_RPS = 624
_TAIL = _N - _NS * _RPS
_ZR = 16
_BM = 1000


def _msg_body(x_hbm, src_hbm, dst_hbm, ea_hbm, w_hbm, agg_hbm,
              src_v, dst_v, attr_v, w_v, rows_a, rows_b, dch_a, dch_b,
              zero_v, agg_sh, g_a, g_b, s_a, s_b):
    c = lax.axis_index("c")
    s = lax.axis_index("s")
    wid = c * _NS + s

    pltpu.sync_copy(w_hbm, w_v)

    zvec = jnp.zeros((16,), jnp.float32)

    def _zrow(r, carry):
        for k in range(8):
            zero_v[r, pl.ds(k * 16, 16)] = zvec
        return carry

    lax.fori_loop(0, _ZR, _zrow, 0)

    def _zcopy(t, carry):
        pltpu.sync_copy(zero_v, agg_sh.at[pl.ds(s * _RPS + t * _ZR, _ZR)])
        return carry

    lax.fori_loop(0, _RPS // _ZR, _zcopy, 0)

    @pl.when(s == _NS - 1)
    def _():
        for t in range(_TAIL // _ZR):
            pltpu.sync_copy(zero_v,
                            agg_sh.at[pl.ds(_NS * _RPS + t * _ZR, _ZR)])

    plsc.subcore_barrier()

    wvec = [[w_v[r, pl.ds(k * 16, 16)] for k in range(8)] for r in range(4)]

    def _gather_start(q, rows, sem):
        pltpu.async_copy(x_hbm.at[src_v.at[pl.ds(q * _CHUNK, _CHUNK)]],
                         rows, sem)

    def _gather_wait(q, rows, sem):
        pltpu.make_async_copy(x_hbm.at[src_v.at[pl.ds(q * _CHUNK, _CHUNK)]],
                              rows, sem).wait()

    def _scat_start(q, rows, sem, dch):
        dch[pl.ds(0, 16)] = dst_v[pl.ds(q * _CHUNK, 16)]
        dch[pl.ds(16, 16)] = dst_v[pl.ds(q * _CHUNK + 16, 16)]
        dch[pl.ds(24, 16)] = dst_v[pl.ds(q * _CHUNK + 24, 16)]
        pltpu.async_copy(rows, agg_sh.at[dch], sem, add=True)

    def _scat_wait(rows, sem, dch):
        pltpu.make_async_copy(rows, agg_sh.at[dch], sem).wait()

    def _compute(q, rows):
        base3 = q * (_CHUNK * 3)

        @plsc.parallel_loop(0, _CHUNK, 1, unroll=2)
        def _edge(e):
            av = attr_v[pl.ds(base3 + e * 3, 16)]
            a0 = av[0]
            a1 = av[1]
            a2 = av[2]
            for k in range(8):
                row = rows[e, pl.ds(k * 16, 16)]
                ea = wvec[3][k] + a0 * wvec[0][k]
                ea = ea + a1 * wvec[1][k]
                ea = ea + a2 * wvec[2][k]
                rows[e, pl.ds(k * 16, 16)] = jnp.maximum(row + ea, 0.0)

    def _group(g, carry):
        gbase = wid * _EPW + g * _GRP
        pltpu.sync_copy(src_hbm.at[pl.ds(gbase, _GRP)], src_v)
        pltpu.sync_copy(dst_hbm.at[pl.ds(gbase, _GRP)], dst_v)
        pltpu.sync_copy(ea_hbm.at[pl.ds(gbase * 3, _GRP * 3)],
                        attr_v.at[pl.ds(0, _GRP * 3)])
        _gather_start(0, rows_a, g_a)

        def _pair(i, icarry):
            q0 = 2 * i
            q1 = q0 + 1

            @pl.when(i > 0)
            def _():
                _scat_wait(rows_b, s_b, dch_b)

            _gather_start(q1, rows_b, g_b)
            _gather_wait(q0, rows_a, g_a)
            _compute(q0, rows_a)
            _scat_start(q0, rows_a, s_a, dch_a)
            _scat_wait(rows_a, s_a, dch_a)

            @pl.when(i < _GPAIR - 1)
            def _():
                _gather_start(q0 + 2, rows_a, g_a)

            _gather_wait(q1, rows_b, g_b)
            _compute(q1, rows_b)
            _scat_start(q1, rows_b, s_b, dch_b)
            return icarry

        lax.fori_loop(0, _GPAIR, _pair, 0)
        _scat_wait(rows_b, s_b, dch_b)
        return carry

    lax.fori_loop(0, _NGRP, _group, 0)
    plsc.subcore_barrier()
    pltpu.sync_copy(agg_sh.at[pl.ds(s * _RPS, _RPS)],
                    agg_hbm.at[c, pl.ds(s * _RPS, _RPS)])

    @pl.when(s == _NS - 1)
    def _():
        pltpu.sync_copy(agg_sh.at[pl.ds(_NS * _RPS, _TAIL)],
                        agg_hbm.at[c, pl.ds(_NS * _RPS, _TAIL)])


_msg = pl.kernel(
    _msg_body,
    out_type=jax.ShapeDtypeStruct((_NC, _N, _D), jnp.float32),
    mesh=plsc.VectorSubcoreMesh(core_axis_name="c", subcore_axis_name="s"),
    scratch_types=[
        pltpu.VMEM((_GRP,), jnp.int32),
        pltpu.VMEM((_GRP,), jnp.int32),
        pltpu.VMEM((_GRP * 3 + 16,), jnp.float32),
        pltpu.VMEM((4, _D), jnp.float32),
        pltpu.VMEM((_CHUNK, _D), jnp.float32),
        pltpu.VMEM((_CHUNK, _D), jnp.float32),
        pltpu.VMEM((_CHUNK,), jnp.int32),
        pltpu.VMEM((_CHUNK,), jnp.int32),
        pltpu.VMEM((_ZR, _D), jnp.float32),
        pltpu.VMEM_SHARED((_N, _D), jnp.float32),
        pltpu.SemaphoreType.DMA,
        pltpu.SemaphoreType.DMA,
        pltpu.SemaphoreType.DMA,
        pltpu.SemaphoreType.DMA,
    ],
)


def _mlp_body(x_ref, a0_ref, a1_ref, wa_ref, ba_ref, wb_ref, bb_ref, o_ref):
    t = x_ref[...] + a0_ref[0] + a1_ref[0]
    u = jnp.dot(t, wa_ref[...], preferred_element_type=jnp.float32)
    u = jnp.maximum(u + ba_ref[...], 0.0)
    h = jnp.dot(u, wb_ref[...], preferred_element_type=jnp.float32)
    o_ref[...] = jnp.maximum(h + bb_ref[...], 0.0)


def _mlp(x, agg, wa, ba, wb, bb):
    return pl.pallas_call(
        _mlp_body,
        grid=(_N // _BM,),
        in_specs=[
            pl.BlockSpec((_BM, _D), lambda i: (i, 0)),
            pl.BlockSpec((1, _BM, _D), lambda i: (0, i, 0)),
            pl.BlockSpec((1, _BM, _D), lambda i: (1, i, 0)),
            pl.BlockSpec((_D, _D), lambda i: (0, 0)),
            pl.BlockSpec((1, _D), lambda i: (0, 0)),
            pl.BlockSpec((_D, _D), lambda i: (0, 0)),
            pl.BlockSpec((1, _D), lambda i: (0, 0)),
        ],
        out_specs=pl.BlockSpec((_BM, _D), lambda i: (i, 0)),
        out_shape=jax.ShapeDtypeStruct((_N, _D), jnp.float32),
    )(x, agg, agg, wa, ba, wb, bb)


def _mlp_pool_body(h_ref, a0_ref, a1_ref, wa_ref, ba_ref, wb_ref, bb_ref,
                   bat_ref, o_ref, s_acc, c_acc):
    i = pl.program_id(0)
    t = h_ref[...] + a0_ref[0] + a1_ref[0]
    u = jnp.dot(t, wa_ref[...], preferred_element_type=jnp.float32)
    u = jnp.maximum(u + ba_ref[...], 0.0)
    h2 = jnp.dot(u, wb_ref[...], preferred_element_type=jnp.float32)
    h2 = jnp.maximum(h2 + bb_ref[...], 0.0)
    onehot = (bat_ref[...] == lax.broadcasted_iota(jnp.int32, (1, _G), 1))
    onehot = onehot.astype(jnp.float32)
    s_part = lax.dot_general(onehot, h2, (((0,), (0,)), ((), ())),
                             preferred_element_type=jnp.float32)
    c_part = lax.dot_general(onehot, jnp.ones((_BM, _D), jnp.float32),
                             (((0,), (0,)), ((), ())),
                             preferred_element_type=jnp.float32)

    @pl.when(i == 0)
    def _():
        s_acc[...] = jnp.zeros_like(s_acc)
        c_acc[...] = jnp.zeros_like(c_acc)

    s_acc[...] += s_part
    c_acc[...] += c_part

    @pl.when(i == pl.num_programs(0) - 1)
    def _():
        o_ref[...] = s_acc[...] / jnp.maximum(c_acc[...], 1.0)


def _mlp_pool(h, agg, wa, ba, wb, bb, batch2):
    return pl.pallas_call(
        _mlp_pool_body,
        grid=(_N // _BM,),
        in_specs=[
            pl.BlockSpec((_BM, _D), lambda i: (i, 0)),
            pl.BlockSpec((1, _BM, _D), lambda i: (0, i, 0)),
            pl.BlockSpec((1, _BM, _D), lambda i: (1, i, 0)),
            pl.BlockSpec((_D, _D), lambda i: (0, 0)),
            pl.BlockSpec((1, _D), lambda i: (0, 0)),
            pl.BlockSpec((_D, _D), lambda i: (0, 0)),
            pl.BlockSpec((1, _D), lambda i: (0, 0)),
            pl.BlockSpec((_BM, 1), lambda i: (i, 0)),
        ],
        out_specs=pl.BlockSpec((_G, _D), lambda i: (0, 0)),
        out_shape=jax.ShapeDtypeStruct((_G, _D), jnp.float32),
        scratch_shapes=[
            pltpu.VMEM((_G, _D), jnp.float32),
            pltpu.VMEM((_G, _D), jnp.float32),
        ],
    )(h, agg, agg, wa, ba, wb, bb, batch2)


def kernel(x, edge_index, edge_attr, batch,
           W1e, b1e, W1a, b1a, W1b, b1b,
           W2e, b2e, W2a, b2a, W2b, b2b):
    src = edge_index[0].astype(jnp.int32)
    dst = edge_index[1].astype(jnp.int32)
    w1 = jnp.concatenate([W1e, b1e[None, :]], axis=0)
    w2 = jnp.concatenate([W2e, b2e[None, :]], axis=0)
    batch2 = batch.reshape(_N, 1).astype(jnp.int32)

    src, dst, ea_flat = lax.optimization_barrier(
        (src, dst, edge_attr.reshape(-1)))

    agg1 = _msg(x, src, dst, ea_flat, w1)
    h1 = _mlp(x, agg1, W1a, b1a.reshape(1, _D), W1b, b1b.reshape(1, _D))
    agg2 = _msg(h1, src, dst, ea_flat, w2)
    return _mlp_pool(h1, agg2, W2a, b2a.reshape(1, _D),
                     W2b, b2b.reshape(1, _D), batch2)


# speedup vs baseline: 1.6442x; 1.3097x over previous
import functools

import jax
import jax.numpy as jnp
from jax import lax
from jax.experimental import pallas as pl
from jax.experimental.pallas import tpu as pltpu
from jax.experimental.pallas import tpu_sc as plsc

_N = 10000
_D = 128
_E = 320000
_G = 64

_NC = 2
_NS = 16
_NW = _NC * _NS
_EPW = _E // _NW
_CHUNK = 40
_GRP = 2000
_NGRP = _EPW // _GRP
_GCH = _GRP // _CHUNK
_GPAIR = _GCH // 2
_RPS = 624
_TAIL = _N - _NS * _RPS
_ZR = 16
_BM = 1000


def _msg_body(x_hbm, src_hbm, dst_hbm, ea_hbm, w_hbm, agg_hbm,
              src_v, dst_v, attr0_v, attr1_v, attr2_v, w_v, rows_a, rows_b,
              dch_a, dch_b, zero_v, agg_sh, g_a, g_b, s_a, s_b):
    c = lax.axis_index("c")
    s = lax.axis_index("s")
    wid = c * _NS + s

    pltpu.sync_copy(w_hbm, w_v)

    zvec = jnp.zeros((16,), jnp.float32)

    def _zrow(r, carry):
        for k in range(8):
            zero_v[r, pl.ds(k * 16, 16)] = zvec
        return carry

    lax.fori_loop(0, _ZR, _zrow, 0)

    def _zcopy(t, carry):
        pltpu.sync_copy(zero_v, agg_sh.at[pl.ds(s * _RPS + t * _ZR, _ZR)])
        return carry

    lax.fori_loop(0, _RPS // _ZR, _zcopy, 0)

    @pl.when(s == _NS - 1)
    def _():
        for t in range(_TAIL // _ZR):
            pltpu.sync_copy(zero_v,
                            agg_sh.at[pl.ds(_NS * _RPS + t * _ZR, _ZR)])

    plsc.subcore_barrier()

    wvec = [[w_v[r, pl.ds(k * 16, 16)] for k in range(8)] for r in range(4)]

    def _gather_start(q, rows, sem):
        pltpu.async_copy(x_hbm.at[src_v.at[pl.ds(q * _CHUNK, _CHUNK)]],
                         rows, sem)

    def _gather_wait(q, rows, sem):
        pltpu.make_async_copy(x_hbm.at[src_v.at[pl.ds(q * _CHUNK, _CHUNK)]],
                              rows, sem).wait()

    def _scat_start(q, rows, sem, dch):
        dch[pl.ds(0, 16)] = dst_v[pl.ds(q * _CHUNK, 16)]
        dch[pl.ds(16, 16)] = dst_v[pl.ds(q * _CHUNK + 16, 16)]
        dch[pl.ds(24, 16)] = dst_v[pl.ds(q * _CHUNK + 24, 16)]
        pltpu.async_copy(rows, agg_sh.at[dch], sem, add=True)

    def _scat_wait(rows, sem, dch):
        pltpu.make_async_copy(rows, agg_sh.at[dch], sem).wait()

    def _compute(q, rows):
        base3 = q * _CHUNK

        @plsc.parallel_loop(0, _CHUNK, 1, unroll=2)
        def _edge(e):
            a0 = attr0_v[pl.ds(base3 + e, 16)][0]
            a1 = attr1_v[pl.ds(base3 + e, 16)][0]
            a2 = attr2_v[pl.ds(base3 + e, 16)][0]
            for k in range(8):
                row = rows[e, pl.ds(k * 16, 16)]
                ea = wvec[3][k] + a0 * wvec[0][k]
                ea = ea + a1 * wvec[1][k]
                ea = ea + a2 * wvec[2][k]
                rows[e, pl.ds(k * 16, 16)] = jnp.maximum(row + ea, 0.0)

    def _group(g, carry):
        gbase = wid * _EPW + g * _GRP
        pltpu.sync_copy(src_hbm.at[pl.ds(gbase, _GRP)], src_v)
        pltpu.sync_copy(dst_hbm.at[pl.ds(gbase, _GRP)], dst_v)
        pltpu.sync_copy(ea_hbm.at[pl.ds(gbase, _GRP)],
                        attr0_v.at[pl.ds(0, _GRP)])
        pltpu.sync_copy(ea_hbm.at[pl.ds(_E + gbase, _GRP)],
                        attr1_v.at[pl.ds(0, _GRP)])
        pltpu.sync_copy(ea_hbm.at[pl.ds(2 * _E + gbase, _GRP)],
                        attr2_v.at[pl.ds(0, _GRP)])
        _gather_start(0, rows_a, g_a)

        def _pair(i, icarry):
            q0 = 2 * i
            q1 = q0 + 1

            @pl.when(i > 0)
            def _():
                _scat_wait(rows_b, s_b, dch_b)

            _gather_start(q1, rows_b, g_b)
            _gather_wait(q0, rows_a, g_a)
            _compute(q0, rows_a)
            _scat_start(q0, rows_a, s_a, dch_a)
            _scat_wait(rows_a, s_a, dch_a)

            @pl.when(i < _GPAIR - 1)
            def _():
                _gather_start(q0 + 2, rows_a, g_a)

            _gather_wait(q1, rows_b, g_b)
            _compute(q1, rows_b)
            _scat_start(q1, rows_b, s_b, dch_b)
            return icarry

        lax.fori_loop(0, _GPAIR, _pair, 0)
        _scat_wait(rows_b, s_b, dch_b)
        return carry

    lax.fori_loop(0, _NGRP, _group, 0)
    plsc.subcore_barrier()
    pltpu.sync_copy(agg_sh.at[pl.ds(s * _RPS, _RPS)],
                    agg_hbm.at[c, pl.ds(s * _RPS, _RPS)])

    @pl.when(s == _NS - 1)
    def _():
        pltpu.sync_copy(agg_sh.at[pl.ds(_NS * _RPS, _TAIL)],
                        agg_hbm.at[c, pl.ds(_NS * _RPS, _TAIL)])


_msg = pl.kernel(
    _msg_body,
    out_type=jax.ShapeDtypeStruct((_NC, _N, _D), jnp.float32),
    mesh=plsc.VectorSubcoreMesh(core_axis_name="c", subcore_axis_name="s"),
    scratch_types=[
        pltpu.VMEM((_GRP,), jnp.int32),
        pltpu.VMEM((_GRP,), jnp.int32),
        pltpu.VMEM((_GRP + 16,), jnp.float32),
        pltpu.VMEM((_GRP + 16,), jnp.float32),
        pltpu.VMEM((_GRP + 16,), jnp.float32),
        pltpu.VMEM((4, _D), jnp.float32),
        pltpu.VMEM((_CHUNK, _D), jnp.float32),
        pltpu.VMEM((_CHUNK, _D), jnp.float32),
        pltpu.VMEM((_CHUNK,), jnp.int32),
        pltpu.VMEM((_CHUNK,), jnp.int32),
        pltpu.VMEM((_ZR, _D), jnp.float32),
        pltpu.VMEM_SHARED((_N, _D), jnp.float32),
        pltpu.SemaphoreType.DMA,
        pltpu.SemaphoreType.DMA,
        pltpu.SemaphoreType.DMA,
        pltpu.SemaphoreType.DMA,
    ],
)


def _mlp_body(x_ref, a0_ref, a1_ref, wa_ref, ba_ref, wb_ref, bb_ref, o_ref):
    t = x_ref[...] + a0_ref[0] + a1_ref[0]
    u = jnp.dot(t, wa_ref[...], preferred_element_type=jnp.float32)
    u = jnp.maximum(u + ba_ref[...], 0.0)
    h = jnp.dot(u, wb_ref[...], preferred_element_type=jnp.float32)
    o_ref[...] = jnp.maximum(h + bb_ref[...], 0.0)


def _mlp(x, agg, wa, ba, wb, bb):
    return pl.pallas_call(
        _mlp_body,
        grid=(_N // _BM,),
        in_specs=[
            pl.BlockSpec((_BM, _D), lambda i: (i, 0)),
            pl.BlockSpec((1, _BM, _D), lambda i: (0, i, 0)),
            pl.BlockSpec((1, _BM, _D), lambda i: (1, i, 0)),
            pl.BlockSpec((_D, _D), lambda i: (0, 0)),
            pl.BlockSpec((1, _D), lambda i: (0, 0)),
            pl.BlockSpec((_D, _D), lambda i: (0, 0)),
            pl.BlockSpec((1, _D), lambda i: (0, 0)),
        ],
        out_specs=pl.BlockSpec((_BM, _D), lambda i: (i, 0)),
        out_shape=jax.ShapeDtypeStruct((_N, _D), jnp.float32),
    )(x, agg, agg, wa, ba, wb, bb)


def _mlp_pool_body(h_ref, a0_ref, a1_ref, wa_ref, ba_ref, wb_ref, bb_ref,
                   bat_ref, o_ref, s_acc, c_acc):
    i = pl.program_id(0)
    t = h_ref[...] + a0_ref[0] + a1_ref[0]
    u = jnp.dot(t, wa_ref[...], preferred_element_type=jnp.float32)
    u = jnp.maximum(u + ba_ref[...], 0.0)
    h2 = jnp.dot(u, wb_ref[...], preferred_element_type=jnp.float32)
    h2 = jnp.maximum(h2 + bb_ref[...], 0.0)
    onehot = (bat_ref[...] == lax.broadcasted_iota(jnp.int32, (1, _G), 1))
    onehot = onehot.astype(jnp.float32)
    s_part = lax.dot_general(onehot, h2, (((0,), (0,)), ((), ())),
                             preferred_element_type=jnp.float32)
    c_part = lax.dot_general(onehot, jnp.ones((_BM, _D), jnp.float32),
                             (((0,), (0,)), ((), ())),
                             preferred_element_type=jnp.float32)

    @pl.when(i == 0)
    def _():
        s_acc[...] = jnp.zeros_like(s_acc)
        c_acc[...] = jnp.zeros_like(c_acc)

    s_acc[...] += s_part
    c_acc[...] += c_part

    @pl.when(i == pl.num_programs(0) - 1)
    def _():
        o_ref[...] = s_acc[...] / jnp.maximum(c_acc[...], 1.0)


def _mlp_pool(h, agg, wa, ba, wb, bb, batch2):
    return pl.pallas_call(
        _mlp_pool_body,
        grid=(_N // _BM,),
        in_specs=[
            pl.BlockSpec((_BM, _D), lambda i: (i, 0)),
            pl.BlockSpec((1, _BM, _D), lambda i: (0, i, 0)),
            pl.BlockSpec((1, _BM, _D), lambda i: (1, i, 0)),
            pl.BlockSpec((_D, _D), lambda i: (0, 0)),
            pl.BlockSpec((1, _D), lambda i: (0, 0)),
            pl.BlockSpec((_D, _D), lambda i: (0, 0)),
            pl.BlockSpec((1, _D), lambda i: (0, 0)),
            pl.BlockSpec((_BM, 1), lambda i: (i, 0)),
        ],
        out_specs=pl.BlockSpec((_G, _D), lambda i: (0, 0)),
        out_shape=jax.ShapeDtypeStruct((_G, _D), jnp.float32),
        scratch_shapes=[
            pltpu.VMEM((_G, _D), jnp.float32),
            pltpu.VMEM((_G, _D), jnp.float32),
        ],
    )(h, agg, agg, wa, ba, wb, bb, batch2)


def kernel(x, edge_index, edge_attr, batch,
           W1e, b1e, W1a, b1a, W1b, b1b,
           W2e, b2e, W2a, b2a, W2b, b2b):
    src = edge_index[0].astype(jnp.int32)
    dst = edge_index[1].astype(jnp.int32)
    w1 = jnp.concatenate([W1e, b1e[None, :]], axis=0)
    w2 = jnp.concatenate([W2e, b2e[None, :]], axis=0)
    batch2 = batch.reshape(_N, 1).astype(jnp.int32)

    src, dst, ea_flat = lax.optimization_barrier(
        (src, dst, edge_attr.T.reshape(-1)))

    agg1 = _msg(x, src, dst, ea_flat, w1)
    h1 = _mlp(x, agg1, W1a, b1a.reshape(1, _D), W1b, b1b.reshape(1, _D))
    agg2 = _msg(h1, src, dst, ea_flat, w2)
    return _mlp_pool(h1, agg2, W2a, b2a.reshape(1, _D),
                     W2b, b2b.reshape(1, _D), batch2)
